# baseline (speedup 1.0000x reference)
import functools

import jax
import jax.numpy as jnp
from jax import lax
from jax.experimental import pallas as pl
from jax.experimental.pallas import tpu as pltpu
from jax.experimental.pallas import tpu_sc as plsc

E = 8
TOPK = 2
H = 2048
F = 2048
FT = 512
NF = F // FT
T = 64
LANES = 16


def _coef_sc_body(te0_hbm, te1_hbm, tw0_hbm, tw1_hbm, coef_hbm,
                  te0_v, te1_v, tw0_v, tw1_v, coef_v):
    cid = lax.axis_index("c")
    sid = lax.axis_index("s")

    @pl.when((cid == 0) & (sid == 0))
    def _():
        pltpu.sync_copy(te0_hbm, te0_v)
        pltpu.sync_copy(te1_hbm, te1_v)
        pltpu.sync_copy(tw0_hbm, tw0_v)
        pltpu.sync_copy(tw1_hbm, tw1_v)
        zero = jnp.zeros((LANES,), jnp.float32)
        for j in range(T // LANES):
            sl = pl.ds(j * LANES, LANES)
            t0 = te0_v[sl]
            t1 = te1_v[sl]
            w0 = tw0_v[sl]
            w1 = tw1_v[sl]
            for e in range(E):
                contrib = (jnp.where(t0 == e, w0, zero)
                           + jnp.where(t1 == e, w1, zero))
                coef_v[e, sl] = contrib
        pltpu.sync_copy(coef_v, coef_hbm)


_coef_sc = pl.kernel(
    _coef_sc_body,
    out_type=jax.ShapeDtypeStruct((E, T), jnp.float32),
    mesh=plsc.VectorSubcoreMesh(core_axis_name="c", subcore_axis_name="s"),
    scratch_types=[
        pltpu.VMEM((T,), jnp.int32),
        pltpu.VMEM((T,), jnp.int32),
        pltpu.VMEM((T,), jnp.float32),
        pltpu.VMEM((T,), jnp.float32),
        pltpu.VMEM((E, T), jnp.float32),
    ],
)


def _moe_body(x_ref, w1_ref, v1_ref, w2_ref, out_ref):
    f = pl.program_id(1)

    @pl.when(f == 0)
    def _init():
        out_ref[:] = jnp.zeros_like(out_ref)

    dn = (((1,), (1,)), ((), ()))
    xw = jax.lax.dot_general(x_ref[:], w1_ref[:], dn,
                             preferred_element_type=jnp.float32)
    xv = jax.lax.dot_general(x_ref[:], v1_ref[:], dn,
                             preferred_element_type=jnp.float32)
    inter = xw * jax.nn.sigmoid(xw) * xv

    out_ref[0] += jnp.dot(inter, w2_ref[:], preferred_element_type=jnp.float32)


def _combine_body(parts_ref, coef_ref, out_ref):
    acc = jnp.zeros((T, H), jnp.float32)
    for e in range(E):
        onehot = (lax.broadcasted_iota(jnp.int32, (E, 1), 0) == e
                  ).astype(jnp.float32)
        col = jax.lax.dot_general(coef_ref[:], onehot, (((0,), (0,)), ((), ())),
                                  preferred_element_type=jnp.float32)
        acc = acc + parts_ref[e] * col
    out_ref[:] = acc


def kernel(x, weights, top_weights, top_experts, w1, v1, w2):
    bsz, q_len, hidden = x.shape
    xf = x.reshape(T, hidden)

    coef = _coef_sc(top_experts[:, 0], top_experts[:, 1],
                    top_weights[:, 0], top_weights[:, 1])

    wspec = pl.BlockSpec((FT, H), lambda e, f: (e * NF + f, 0))
    grid = (E, NF)
    parts = pl.pallas_call(
        _moe_body,
        grid=grid,
        in_specs=[
            pl.BlockSpec((T, H), lambda e, f: (0, 0)),
            wspec,
            wspec,
            wspec,
        ],
        out_specs=pl.BlockSpec((1, T, H), lambda e, f: (e, 0, 0)),
        out_shape=jax.ShapeDtypeStruct((E, T, H), jnp.float32),
        compiler_params=pltpu.CompilerParams(
            dimension_semantics=("arbitrary", "arbitrary"),
        ),
    )(xf, w1, v1, w2)

    out = pl.pallas_call(
        _combine_body,
        out_shape=jax.ShapeDtypeStruct((T, H), jnp.float32),
    )(parts, coef)
    return out.reshape(bsz, q_len, hidden)


# speedup vs baseline: 1.1437x; 1.1437x over previous
import functools

import jax
import jax.numpy as jnp
from jax.experimental import pallas as pl
from jax.experimental.pallas import tpu as pltpu

E = 8
TOPK = 2
H = 2048
F = 2048
FT = 512
NF = F // FT


def _moe_body(x_ref, tw_ref, te_ref, w1_ref, v1_ref, w2_ref, out_ref):
    e = pl.program_id(0)
    f = pl.program_id(1)

    @pl.when((e == 0) & (f == 0))
    def _init():
        out_ref[:] = jnp.zeros_like(out_ref)

    dn = (((1,), (1,)), ((), ()))
    xw = jax.lax.dot_general(x_ref[:], w1_ref[:], dn,
                             preferred_element_type=jnp.float32)
    xv = jax.lax.dot_general(x_ref[:], v1_ref[:], dn,
                             preferred_element_type=jnp.float32)
    inter = xw * jax.nn.sigmoid(xw) * xv

    sel = te_ref[:] == e
    coef = jnp.sum(jnp.where(sel, tw_ref[:], 0.0), axis=-1)
    inter = inter * coef[:, None]

    out_ref[:] += jnp.dot(inter, w2_ref[:], preferred_element_type=jnp.float32)


def kernel(x, weights, top_weights, top_experts, w1, v1, w2):
    bsz, q_len, hidden = x.shape
    T = bsz * q_len
    xf = x.reshape(T, hidden)

    wspec = pl.BlockSpec((FT, H), lambda e, f: (e * NF + f, 0))
    grid = (E, NF)
    out = pl.pallas_call(
        _moe_body,
        grid=grid,
        in_specs=[
            pl.BlockSpec((T, H), lambda e, f: (0, 0)),
            pl.BlockSpec((T, TOPK), lambda e, f: (0, 0)),
            pl.BlockSpec((T, TOPK), lambda e, f: (0, 0)),
            wspec,
            wspec,
            wspec,
        ],
        out_specs=pl.BlockSpec((T, H), lambda e, f: (0, 0)),
        out_shape=jax.ShapeDtypeStruct((T, H), jnp.float32),
        compiler_params=pltpu.CompilerParams(
            dimension_semantics=("arbitrary", "arbitrary"),
        ),
    )(xf, top_weights, top_experts, w1, v1, w2)
    return out.reshape(bsz, q_len, hidden)


# speedup vs baseline: 1.1936x; 1.0437x over previous
import functools

import jax
import jax.numpy as jnp
from jax.experimental import pallas as pl
from jax.experimental.pallas import tpu as pltpu

E = 8
TOPK = 2
H = 2048
F = 2048
FT = 512
NF = F // FT


def _moe_body(x_ref, tw_ref, te_ref, w1_ref, v1_ref, w2_ref, out_ref):
    e = pl.program_id(0)
    f = pl.program_id(1)

    @pl.when((e == 0) & (f == 0))
    def _init():
        out_ref[:] = jnp.zeros_like(out_ref)

    dn = (((1,), (1,)), ((), ()))
    x = x_ref[:, 0, :]
    xw = jax.lax.dot_general(x, w1_ref[:], dn,
                             preferred_element_type=jnp.float32)
    xv = jax.lax.dot_general(x, v1_ref[:], dn,
                             preferred_element_type=jnp.float32)
    inter = xw * jax.nn.sigmoid(xw) * xv

    sel = te_ref[:] == e
    coef = jnp.sum(jnp.where(sel, tw_ref[:], 0.0), axis=-1)
    inter = inter * coef[:, None]

    out_ref[:, 0, :] += jnp.dot(inter, w2_ref[:],
                                preferred_element_type=jnp.float32)


def kernel(x, weights, top_weights, top_experts, w1, v1, w2):
    bsz, q_len, hidden = x.shape
    T = bsz * q_len

    wspec = pl.BlockSpec((FT, H), lambda e, f: (e * NF + f, 0))
    grid = (E, NF)
    out = pl.pallas_call(
        _moe_body,
        grid=grid,
        in_specs=[
            pl.BlockSpec((bsz, q_len, H), lambda e, f: (0, 0, 0)),
            pl.BlockSpec((T, TOPK), lambda e, f: (0, 0)),
            pl.BlockSpec((T, TOPK), lambda e, f: (0, 0)),
            wspec,
            wspec,
            wspec,
        ],
        out_specs=pl.BlockSpec((bsz, q_len, H), lambda e, f: (0, 0, 0)),
        out_shape=jax.ShapeDtypeStruct((bsz, q_len, H), jnp.float32),
        compiler_params=pltpu.CompilerParams(
            dimension_semantics=("arbitrary", "arbitrary"),
        ),
    )(x, top_weights, top_experts, w1, v1, w2)
    return out


# speedup vs baseline: 1.1965x; 1.0024x over previous
import functools

import jax
import jax.numpy as jnp
from jax.experimental import pallas as pl
from jax.experimental.pallas import tpu as pltpu

E = 8
TOPK = 2
H = 2048
F = 2048
FT = 512
NF = F // FT


def _moe_body(x_ref, tw_ref, te_ref, w1_ref, v1_ref, w2_ref, out_ref):
    e = pl.program_id(0)
    f = pl.program_id(1)

    @pl.when((e == 0) & (f == 0))
    def _init():
        out_ref[:] = jnp.zeros_like(out_ref)

    dn = (((1,), (1,)), ((), ()))
    x = x_ref[:, 0, :]
    xw = jax.lax.dot_general(x, w1_ref[:], dn,
                             preferred_element_type=jnp.float32)
    xv = jax.lax.dot_general(x, v1_ref[:], dn,
                             preferred_element_type=jnp.float32)
    inter = xw * jax.nn.sigmoid(xw) * xv

    sel = te_ref[:] == e
    coef = jnp.sum(jnp.where(sel, tw_ref[:], 0.0), axis=-1)
    inter = inter * coef[:, None]

    out_ref[:, 0, :] += jnp.dot(inter, w2_ref[:],
                                preferred_element_type=jnp.float32)


def kernel(x, weights, top_weights, top_experts, w1, v1, w2):
    bsz, q_len, hidden = x.shape
    T = bsz * q_len

    wspec = pl.BlockSpec((FT, H), lambda e, f: (e * NF + f, 0))
    grid = (E, NF)
    out = pl.pallas_call(
        _moe_body,
        grid=grid,
        in_specs=[
            pl.BlockSpec((bsz, q_len, H), lambda e, f: (0, 0, 0)),
            pl.BlockSpec((T, TOPK), lambda e, f: (0, 0)),
            pl.BlockSpec((T, TOPK), lambda e, f: (0, 0)),
            wspec,
            wspec,
            wspec,
        ],
        out_specs=pl.BlockSpec((bsz, q_len, H), lambda e, f: (0, 0, 0)),
        out_shape=jax.ShapeDtypeStruct((bsz, q_len, H), jnp.float32),
        compiler_params=pltpu.CompilerParams(
            dimension_semantics=("arbitrary", "arbitrary"),
            vmem_limit_bytes=120 * 1024 * 1024,
        ),
    )(x, top_weights, top_experts, w1, v1, w2)
    return out
